# baseline (speedup 1.0000x reference)
import functools

import jax
import jax.numpy as jnp
from jax.experimental import pallas as pl
from jax.experimental.pallas import tpu as pltpu

_MB = 4


def _conv_stage_kernel(x_ref, w_ref, b_ref, sc_ref, sh_ref,
                       y_ref, sum_ref, sq_ref, buf_ref,
                       *, H, W, K, affine):
    assert K == 3, "flat-shift tap scheme is written for 3x3"
    MB = x_ref.shape[0]
    HW = H * W
    C = x_ref.shape[-1]
    S = buf_ref.shape[0]

    col = jax.lax.broadcasted_iota(jnp.int32, (HW, 1), 0) % W
    m0 = (col != 0).astype(jnp.float32)
    m2 = (col != W - 1).astype(jnp.float32)
    Cout = b_ref.shape[1]

    buf_ref[0:W] = jnp.zeros((W, C), jnp.bfloat16)
    buf_ref[W + HW:] = jnp.zeros((S - W - HW, C), jnp.bfloat16)

    s_tot = None
    q_tot = None
    for b in range(MB):
        x = x_ref[b].reshape(HW, C).astype(jnp.float32)
        if affine:
            x = x * sc_ref[...] + sh_ref[...]
        buf_ref[W:W + HW] = x.astype(jnp.bfloat16)

        z = []
        for kw in range(K):
            zk = None
            for kh in range(K):
                lhs = buf_ref[kh * W:kh * W + HW, :]
                d = jnp.dot(lhs, w_ref[kh * K + kw],
                            preferred_element_type=jnp.float32)
                zk = d if zk is None else zk + d
            z.append(zk)

        zf = jnp.zeros((1, Cout), jnp.float32)
        sd = jnp.concatenate([zf, z[0][:HW - 1]], axis=0) * m0
        su = jnp.concatenate([z[2][1:], zf], axis=0) * m2
        y = jnp.maximum(z[1] + sd + su + b_ref[...], 0.0)
        s = jnp.sum(y, axis=0, keepdims=True)
        q = jnp.sum(y * y, axis=0, keepdims=True)
        s_tot = s if s_tot is None else s_tot + s
        q_tot = q if q_tot is None else q_tot + q
        y_ref[b] = y.astype(y_ref.dtype)

    sum_ref[0] = s_tot
    sq_ref[0] = q_tot


def _conv_stage(x, w3, b, sc, sh, H, W, affine):
    N = x.shape[0]
    KK, C, Cout = w3.shape
    K = int(round(KK ** 0.5))
    p = (K - 1) // 2
    HW = H * W
    MB = _MB if N % _MB == 0 else 1
    G = N // MB
    S = (H + 2 * p) * W

    if x.ndim == 4:
        x_spec = pl.BlockSpec((MB, H, W, C), lambda n: (n, 0, 0, 0))
    else:
        x_spec = pl.BlockSpec((MB, HW, C), lambda n: (n, 0, 0))

    kern = functools.partial(_conv_stage_kernel, H=H, W=W, K=K, affine=affine)
    return pl.pallas_call(
        kern,
        grid=(G,),
        out_shape=(
            jax.ShapeDtypeStruct((N, HW, Cout), jnp.bfloat16),
            jax.ShapeDtypeStruct((G, 1, Cout), jnp.float32),
            jax.ShapeDtypeStruct((G, 1, Cout), jnp.float32),
        ),
        in_specs=[
            x_spec,
            pl.BlockSpec((KK, C, Cout), lambda n: (0, 0, 0)),
            pl.BlockSpec((1, Cout), lambda n: (0, 0)),
            pl.BlockSpec((1, C), lambda n: (0, 0)),
            pl.BlockSpec((1, C), lambda n: (0, 0)),
        ],
        out_specs=(
            pl.BlockSpec((MB, HW, Cout), lambda n: (n, 0, 0)),
            pl.BlockSpec((1, 1, Cout), lambda n: (n, 0, 0)),
            pl.BlockSpec((1, 1, Cout), lambda n: (n, 0, 0)),
        ),
        scratch_shapes=[
            pltpu.VMEM((S, C), jnp.bfloat16),
        ],
        compiler_params=pltpu.CompilerParams(
            dimension_semantics=("parallel",),
            vmem_limit_bytes=48 * 1024 * 1024,
        ),
    )(x, w3, b, sc, sh)


def _affine_nchw_kernel(y_ref, sc_ref, sh_ref, o_ref):
    for b in range(y_ref.shape[0]):
        y = y_ref[b].astype(jnp.float32) * sc_ref[...] + sh_ref[...]
        o_ref[b] = jnp.transpose(y).astype(o_ref.dtype)


def _apply_affine_nchw(y, sc, sh, out_dtype):
    N, HW, C = y.shape
    MB = _MB if N % _MB == 0 else 1
    G = N // MB
    return pl.pallas_call(
        _affine_nchw_kernel,
        grid=(G,),
        out_shape=jax.ShapeDtypeStruct((N, C, HW), out_dtype),
        in_specs=[
            pl.BlockSpec((MB, HW, C), lambda n: (n, 0, 0)),
            pl.BlockSpec((1, C), lambda n: (0, 0)),
            pl.BlockSpec((1, C), lambda n: (0, 0)),
        ],
        out_specs=pl.BlockSpec((MB, C, HW), lambda n: (n, 0, 0)),
        compiler_params=pltpu.CompilerParams(
            dimension_semantics=("parallel",),
        ),
    )(y, sc, sh)


def _bn_affine(part_sum, part_sq, gamma, beta, count, eps):
    s = jnp.sum(part_sum[:, 0, :], axis=0)
    q = jnp.sum(part_sq[:, 0, :], axis=0)
    mean = s / count
    var = jnp.maximum(q / count - mean * mean, 0.0)
    inv = jax.lax.rsqrt(var + eps)
    scale = gamma.astype(jnp.float32) * inv
    shift = beta.astype(jnp.float32) - mean * scale
    C = scale.shape[0]
    return scale.reshape(1, C), shift.reshape(1, C)


def kernel(x, w1, b1, g1, be1, w2, b2, g2, be2):
    N, Cin, H, W = x.shape
    K = w1.shape[0]
    C1 = w1.shape[3]
    C2 = w2.shape[3]
    eps = 1e-5

    x_nhwc = jnp.transpose(x, (0, 2, 3, 1))
    w1b = w1.astype(jnp.bfloat16).reshape(K * K, Cin, C1)
    w2b = w2.astype(jnp.bfloat16).reshape(K * K, C1, C2)
    b1c = b1.astype(jnp.float32).reshape(1, C1)
    b2c = b2.astype(jnp.float32).reshape(1, C2)
    one = jnp.ones((1, Cin), jnp.float32)
    zero = jnp.zeros((1, Cin), jnp.float32)

    y1, s1, q1 = _conv_stage(x_nhwc, w1b, b1c, one, zero, H, W, affine=False)
    sc1, sh1 = _bn_affine(s1, q1, g1, be1, N * H * W, eps)

    y2, s2, q2 = _conv_stage(y1, w2b, b2c, sc1, sh1, H, W, affine=True)
    sc2, sh2 = _bn_affine(s2, q2, g2, be2, N * H * W, eps)

    out = _apply_affine_nchw(y2, sc2, sh2, x.dtype)
    return out.reshape(N, C2, H, W)


# speedup vs baseline: 1.0121x; 1.0121x over previous
import functools

import jax
import jax.numpy as jnp
from jax.experimental import pallas as pl
from jax.experimental.pallas import tpu as pltpu

_MB = 8


def _conv_stage_kernel(x_ref, w_ref, b_ref, sc_ref, sh_ref,
                       y_ref, sum_ref, sq_ref, buf_ref,
                       *, H, W, K, affine):
    assert K == 3, "flat-shift tap scheme is written for 3x3"
    MB = x_ref.shape[0]
    HW = H * W
    C = x_ref.shape[-1]
    S = buf_ref.shape[0]

    col = jax.lax.broadcasted_iota(jnp.int32, (HW, 1), 0) % W
    m0 = (col != 0).astype(jnp.float32)
    m2 = (col != W - 1).astype(jnp.float32)
    Cout = b_ref.shape[1]

    buf_ref[0:W] = jnp.zeros((W, C), jnp.bfloat16)
    buf_ref[W + HW:] = jnp.zeros((S - W - HW, C), jnp.bfloat16)

    s_tot = None
    q_tot = None
    for b in range(MB):
        x = x_ref[b].reshape(HW, C).astype(jnp.float32)
        if affine:
            x = x * sc_ref[...] + sh_ref[...]
        buf_ref[W:W + HW] = x.astype(jnp.bfloat16)

        z = []
        for kw in range(K):
            zk = None
            for kh in range(K):
                lhs = buf_ref[kh * W:kh * W + HW, :]
                d = jnp.dot(lhs, w_ref[kh * K + kw],
                            preferred_element_type=jnp.float32)
                zk = d if zk is None else zk + d
            z.append(zk)

        zf = jnp.zeros((1, Cout), jnp.float32)
        sd = jnp.concatenate([zf, z[0][:HW - 1]], axis=0) * m0
        su = jnp.concatenate([z[2][1:], zf], axis=0) * m2
        y = jnp.maximum(z[1] + sd + su + b_ref[...], 0.0)
        s = jnp.sum(y, axis=0, keepdims=True)
        q = jnp.sum(y * y, axis=0, keepdims=True)
        s_tot = s if s_tot is None else s_tot + s
        q_tot = q if q_tot is None else q_tot + q
        y_ref[b] = y.astype(y_ref.dtype)

    sum_ref[0] = s_tot
    sq_ref[0] = q_tot


def _conv_stage(x, w3, b, sc, sh, H, W, affine):
    N = x.shape[0]
    KK, C, Cout = w3.shape
    K = int(round(KK ** 0.5))
    p = (K - 1) // 2
    HW = H * W
    MB = _MB if N % _MB == 0 else 1
    G = N // MB
    S = (H + 2 * p) * W

    if x.ndim == 4:
        x_spec = pl.BlockSpec((MB, H, W, C), lambda n: (n, 0, 0, 0))
    else:
        x_spec = pl.BlockSpec((MB, HW, C), lambda n: (n, 0, 0))

    kern = functools.partial(_conv_stage_kernel, H=H, W=W, K=K, affine=affine)
    return pl.pallas_call(
        kern,
        grid=(G,),
        out_shape=(
            jax.ShapeDtypeStruct((N, HW, Cout), jnp.bfloat16),
            jax.ShapeDtypeStruct((G, 1, Cout), jnp.float32),
            jax.ShapeDtypeStruct((G, 1, Cout), jnp.float32),
        ),
        in_specs=[
            x_spec,
            pl.BlockSpec((KK, C, Cout), lambda n: (0, 0, 0)),
            pl.BlockSpec((1, Cout), lambda n: (0, 0)),
            pl.BlockSpec((1, C), lambda n: (0, 0)),
            pl.BlockSpec((1, C), lambda n: (0, 0)),
        ],
        out_specs=(
            pl.BlockSpec((MB, HW, Cout), lambda n: (n, 0, 0)),
            pl.BlockSpec((1, 1, Cout), lambda n: (n, 0, 0)),
            pl.BlockSpec((1, 1, Cout), lambda n: (n, 0, 0)),
        ),
        scratch_shapes=[
            pltpu.VMEM((S, C), jnp.bfloat16),
        ],
        compiler_params=pltpu.CompilerParams(
            dimension_semantics=("parallel",),
            vmem_limit_bytes=48 * 1024 * 1024,
        ),
    )(x, w3, b, sc, sh)


def _affine_nchw_kernel(y_ref, sc_ref, sh_ref, o_ref):
    for b in range(y_ref.shape[0]):
        y = y_ref[b].astype(jnp.float32) * sc_ref[...] + sh_ref[...]
        o_ref[b] = jnp.transpose(y).astype(o_ref.dtype)


def _apply_affine_nchw(y, sc, sh, out_dtype):
    N, HW, C = y.shape
    MB = _MB if N % _MB == 0 else 1
    G = N // MB
    return pl.pallas_call(
        _affine_nchw_kernel,
        grid=(G,),
        out_shape=jax.ShapeDtypeStruct((N, C, HW), out_dtype),
        in_specs=[
            pl.BlockSpec((MB, HW, C), lambda n: (n, 0, 0)),
            pl.BlockSpec((1, C), lambda n: (0, 0)),
            pl.BlockSpec((1, C), lambda n: (0, 0)),
        ],
        out_specs=pl.BlockSpec((MB, C, HW), lambda n: (n, 0, 0)),
        compiler_params=pltpu.CompilerParams(
            dimension_semantics=("parallel",),
        ),
    )(y, sc, sh)


def _bn_affine(part_sum, part_sq, gamma, beta, count, eps):
    s = jnp.sum(part_sum[:, 0, :], axis=0)
    q = jnp.sum(part_sq[:, 0, :], axis=0)
    mean = s / count
    var = jnp.maximum(q / count - mean * mean, 0.0)
    inv = jax.lax.rsqrt(var + eps)
    scale = gamma.astype(jnp.float32) * inv
    shift = beta.astype(jnp.float32) - mean * scale
    C = scale.shape[0]
    return scale.reshape(1, C), shift.reshape(1, C)


def kernel(x, w1, b1, g1, be1, w2, b2, g2, be2):
    N, Cin, H, W = x.shape
    K = w1.shape[0]
    C1 = w1.shape[3]
    C2 = w2.shape[3]
    eps = 1e-5

    x_nhwc = jnp.transpose(x, (0, 2, 3, 1))
    w1b = w1.astype(jnp.bfloat16).reshape(K * K, Cin, C1)
    w2b = w2.astype(jnp.bfloat16).reshape(K * K, C1, C2)
    b1c = b1.astype(jnp.float32).reshape(1, C1)
    b2c = b2.astype(jnp.float32).reshape(1, C2)
    one = jnp.ones((1, Cin), jnp.float32)
    zero = jnp.zeros((1, Cin), jnp.float32)

    y1, s1, q1 = _conv_stage(x_nhwc, w1b, b1c, one, zero, H, W, affine=False)
    sc1, sh1 = _bn_affine(s1, q1, g1, be1, N * H * W, eps)

    y2, s2, q2 = _conv_stage(y1, w2b, b2c, sc1, sh1, H, W, affine=True)
    sc2, sh2 = _bn_affine(s2, q2, g2, be2, N * H * W, eps)

    out = _apply_affine_nchw(y2, sc2, sh2, x.dtype)
    return out.reshape(N, C2, H, W)
